# baseline (speedup 1.0000x reference)
import functools

import jax
import jax.numpy as jnp
from jax import lax
from jax.experimental import pallas as pl
from jax.experimental.pallas import tpu as pltpu
from jax.experimental.pallas import tpu_sc as plsc

NN = 10000
EE = 320000
DD = 128
DH = DD // 2
LAYERS = 5
EPS = 0.0
BN_EPS = 1e-5

NC = 2
NS = 16
EPW = EE // NS
CH = 128
K = 3
GRP = 2 * K * CH
NGRP = EPW // GRP
REM = EPW - NGRP * GRP
RPT = 624
TAIL = NN - NS * RPT


def _sc_agg_body(h_hbm, src2_hbm, dst_hbm, zeros_hbm, out_hbm,
                 src_v, dst_v, rows_r, acc_sh, semg, sems):
    c = lax.axis_index("c")
    s = lax.axis_index("s")
    col = pl.multiple_of(c * DH, DH)

    pltpu.sync_copy(zeros_hbm.at[pl.ds(s * RPT, RPT), pl.ds(0, DH)],
                    acc_sh.at[pl.ds(s * RPT, RPT)])
    @pl.when(s == 0)
    def _():
        pltpu.sync_copy(zeros_hbm.at[pl.ds(NS * RPT, TAIL), pl.ds(0, DH)],
                        acc_sh.at[pl.ds(NS * RPT, TAIL)])
    pltpu.sync_copy(src2_hbm.at[pl.ds(c * EE + s * EPW, EPW)], src_v)
    pltpu.sync_copy(dst_hbm.at[pl.ds(s * EPW, EPW)], dst_v)
    plsc.subcore_barrier()

    def gath(o, b):
        return pltpu.async_copy(h_hbm.at[src_v.at[pl.ds(o, CH)]],
                                rows_r.at[b], semg)

    def scat(o, b):
        return pltpu.async_copy(rows_r.at[b],
                                acc_sh.at[dst_v.at[pl.ds(o, CH)]],
                                sems, add=True)

    def wait_scat(o, b):
        pltpu.make_async_copy(rows_r.at[b],
                              acc_sh.at[dst_v.at[pl.ds(o, CH)]],
                              sems).wait()

    def group(i, _):
        base_o = pl.multiple_of(i * GRP, GRP)
        prev_o = base_o - GRP

        @pl.when(i > 0)
        def _():
            for b in range(K):
                wait_scat(prev_o + b * CH, b)
        ga = [gath(base_o + b * CH, b) for b in range(K)]
        for d in ga:
            d.wait()

        @pl.when(i > 0)
        def _():
            for b in range(K, 2 * K):
                wait_scat(prev_o + b * CH, b)
        for b in range(K):
            scat(base_o + b * CH, b)
        gb = [gath(base_o + b * CH, b) for b in range(K, 2 * K)]
        for d in gb:
            d.wait()
        for b in range(K, 2 * K):
            scat(base_o + b * CH, b)
        return 0

    lax.fori_loop(0, NGRP, group, 0)

    last_o = (NGRP - 1) * GRP
    for b in range(2 * K):
        wait_scat(last_o + b * CH, b)

    ox = NGRP * GRP
    pltpu.async_copy(h_hbm.at[src_v.at[pl.ds(ox, REM)]],
                     rows_r.at[0, pl.ds(0, REM)], semg).wait()
    pltpu.async_copy(rows_r.at[0, pl.ds(0, REM)],
                     acc_sh.at[dst_v.at[pl.ds(ox, REM)]], sems,
                     add=True).wait()

    plsc.subcore_barrier()
    pltpu.sync_copy(acc_sh.at[pl.ds(s * RPT, RPT)],
                    out_hbm.at[pl.ds(s * RPT, RPT), pl.ds(col, DH)])
    @pl.when(s == 0)
    def _():
        pltpu.sync_copy(acc_sh.at[pl.ds(NS * RPT, TAIL)],
                        out_hbm.at[pl.ds(NS * RPT, TAIL), pl.ds(col, DH)])


_sc_agg = functools.partial(
    pl.kernel,
    out_type=jax.ShapeDtypeStruct((NN, DD), jnp.float32),
    mesh=plsc.VectorSubcoreMesh(core_axis_name="c", subcore_axis_name="s"),
    compiler_params=pltpu.CompilerParams(use_tc_tiling_on_sc=False),
    scratch_types=[
        pltpu.VMEM((EPW,), jnp.int32),
        pltpu.VMEM((EPW,), jnp.int32),
        pltpu.VMEM((2 * K, CH, DH), jnp.float32),
        pltpu.VMEM_SHARED((NN, DH), jnp.float32),
        pltpu.SemaphoreType.DMA,
        pltpu.SemaphoreType.DMA,
    ],
)(_sc_agg_body)


def _unstack(a):
    return jnp.concatenate([a[:NN], a[NN:]], axis=1)


def _mlp_bn_body(h_ref, a_ref, w1_ref, b1_ref, w2_ref, b2_ref,
                 g_ref, be_ref, o_ref):
    x = _unstack(h_ref[...]) * (1.0 + EPS) + a_ref[...]
    t = jnp.maximum(
        jnp.dot(x, w1_ref[...], preferred_element_type=jnp.float32)
        + b1_ref[...], 0.0)
    y = (jnp.dot(t, w2_ref[...], preferred_element_type=jnp.float32)
         + b2_ref[...])
    mu = jnp.mean(y, axis=0, keepdims=True)
    var = jnp.mean((y - mu) ** 2, axis=0, keepdims=True)
    yn = g_ref[...] * (y - mu) * lax.rsqrt(var + BN_EPS) + be_ref[...]
    yn = jnp.maximum(yn, 0.0)
    o_ref[...] = jnp.concatenate([yn[:, :DH], yn[:, DH:]], axis=0)


def _mlp_final_body(h_ref, a_ref, w1_ref, b1_ref, w2_ref, b2_ref, o_ref):
    x = _unstack(h_ref[...]) * (1.0 + EPS) + a_ref[...]
    t = jnp.maximum(
        jnp.dot(x, w1_ref[...], preferred_element_type=jnp.float32)
        + b1_ref[...], 0.0)
    o_ref[...] = (jnp.dot(t, w2_ref[...], preferred_element_type=jnp.float32)
                  + b2_ref[...])


_mlp_bn = pl.pallas_call(
    _mlp_bn_body,
    out_shape=jax.ShapeDtypeStruct((2 * NN, DH), jnp.float32),
)

_mlp_final = pl.pallas_call(
    _mlp_final_body,
    out_shape=jax.ShapeDtypeStruct((NN, DD), jnp.float32),
)


def kernel(x, edge_index, params):
    src = edge_index[0]
    dst = edge_index[1]
    src2 = jnp.concatenate([src, src + NN])
    zeros = jnp.zeros((NN, DD), jnp.float32)
    h = jnp.concatenate([x[:, :DH], x[:, DH:]], axis=0)
    for i in range(LAYERS):
        W1, b1, W2, b2 = params["convs"][i]
        agg = _sc_agg(h, src2, dst, zeros)
        b1r = b1.reshape(1, DD)
        b2r = b2.reshape(1, DD)
        if i < LAYERS - 1:
            gamma, beta = params["bns"][i]
            h = _mlp_bn(h, agg, W1, b1r, W2, b2r,
                        gamma.reshape(1, DD), beta.reshape(1, DD))
        else:
            h = _mlp_final(h, agg, W1, b1r, W2, b2r)
    return h


# speedup vs baseline: 1.0223x; 1.0223x over previous
import functools

import jax
import jax.numpy as jnp
from jax import lax
from jax.experimental import pallas as pl
from jax.experimental.pallas import tpu as pltpu
from jax.experimental.pallas import tpu_sc as plsc

NN = 10000
EE = 320000
DD = 128
DH = DD // 2
LAYERS = 5
EPS = 0.0
BN_EPS = 1e-5

NC = 2
NS = 16
EPW = EE // NS
CH = 128
K = 3
GRP = 2 * K * CH
NGRP = EPW // GRP
REM = EPW - NGRP * GRP
RPT = 624
TAIL = NN - NS * RPT


def _sc_agg_body(h_hbm, src2_hbm, dst_hbm, zeros_hbm, out_hbm,
                 src_v, dst_v, rows_r, acc_sh, semg, sems):
    c = lax.axis_index("c")
    s = lax.axis_index("s")
    col = pl.multiple_of(c * DH, DH)

    pre = [
        pltpu.async_copy(zeros_hbm.at[pl.ds(s * RPT, RPT), pl.ds(0, DH)],
                         acc_sh.at[pl.ds(s * RPT, RPT)], semg),
        pltpu.async_copy(src2_hbm.at[pl.ds(c * EE + s * EPW, EPW)], src_v,
                         sems),
        pltpu.async_copy(dst_hbm.at[pl.ds(s * EPW, EPW)], dst_v, sems),
    ]
    @pl.when(s == 0)
    def _():
        pltpu.sync_copy(zeros_hbm.at[pl.ds(NS * RPT, TAIL), pl.ds(0, DH)],
                        acc_sh.at[pl.ds(NS * RPT, TAIL)])
    for d in pre:
        d.wait()
    plsc.subcore_barrier()

    def gath(o, b):
        return pltpu.async_copy(h_hbm.at[src_v.at[pl.ds(o, CH)]],
                                rows_r.at[b], semg)

    def scat(o, b):
        return pltpu.async_copy(rows_r.at[b],
                                acc_sh.at[dst_v.at[pl.ds(o, CH)]],
                                sems, add=True)

    def wait_scat(o, b):
        pltpu.make_async_copy(rows_r.at[b],
                              acc_sh.at[dst_v.at[pl.ds(o, CH)]],
                              sems).wait()

    def group(i, _):
        base_o = pl.multiple_of(i * GRP, GRP)
        prev_o = base_o - GRP

        @pl.when(i > 0)
        def _():
            for b in range(K):
                wait_scat(prev_o + b * CH, b)
        ga = [gath(base_o + b * CH, b) for b in range(K)]
        for d in ga:
            d.wait()

        @pl.when(i > 0)
        def _():
            for b in range(K, 2 * K):
                wait_scat(prev_o + b * CH, b)
        for b in range(K):
            scat(base_o + b * CH, b)
        gb = [gath(base_o + b * CH, b) for b in range(K, 2 * K)]
        for d in gb:
            d.wait()
        for b in range(K, 2 * K):
            scat(base_o + b * CH, b)
        return 0

    lax.fori_loop(0, NGRP, group, 0)

    last_o = (NGRP - 1) * GRP
    for b in range(2 * K):
        wait_scat(last_o + b * CH, b)

    ox = NGRP * GRP
    pltpu.async_copy(h_hbm.at[src_v.at[pl.ds(ox, REM)]],
                     rows_r.at[0, pl.ds(0, REM)], semg).wait()
    pltpu.async_copy(rows_r.at[0, pl.ds(0, REM)],
                     acc_sh.at[dst_v.at[pl.ds(ox, REM)]], sems,
                     add=True).wait()

    plsc.subcore_barrier()
    pltpu.sync_copy(acc_sh.at[pl.ds(s * RPT, RPT)],
                    out_hbm.at[pl.ds(s * RPT, RPT), pl.ds(col, DH)])
    @pl.when(s == 0)
    def _():
        pltpu.sync_copy(acc_sh.at[pl.ds(NS * RPT, TAIL)],
                        out_hbm.at[pl.ds(NS * RPT, TAIL), pl.ds(col, DH)])


_sc_agg = functools.partial(
    pl.kernel,
    out_type=jax.ShapeDtypeStruct((NN, DD), jnp.float32),
    mesh=plsc.VectorSubcoreMesh(core_axis_name="c", subcore_axis_name="s"),
    compiler_params=pltpu.CompilerParams(use_tc_tiling_on_sc=False),
    scratch_types=[
        pltpu.VMEM((EPW,), jnp.int32),
        pltpu.VMEM((EPW,), jnp.int32),
        pltpu.VMEM((2 * K, CH, DH), jnp.float32),
        pltpu.VMEM_SHARED((NN, DH), jnp.float32),
        pltpu.SemaphoreType.DMA,
        pltpu.SemaphoreType.DMA,
    ],
)(_sc_agg_body)


def _unstack(a):
    return jnp.concatenate([a[:NN], a[NN:]], axis=1)


def _mlp_bn_body(h_ref, a_ref, w1_ref, b1_ref, w2_ref, b2_ref,
                 g_ref, be_ref, o_ref):
    x = _unstack(h_ref[...]) * (1.0 + EPS) + a_ref[...]
    t = jnp.maximum(
        jnp.dot(x, w1_ref[...], preferred_element_type=jnp.float32)
        + b1_ref[...], 0.0)
    y = (jnp.dot(t, w2_ref[...], preferred_element_type=jnp.float32)
         + b2_ref[...])
    mu = jnp.mean(y, axis=0, keepdims=True)
    var = jnp.mean((y - mu) ** 2, axis=0, keepdims=True)
    yn = g_ref[...] * (y - mu) * lax.rsqrt(var + BN_EPS) + be_ref[...]
    yn = jnp.maximum(yn, 0.0)
    o_ref[...] = jnp.concatenate([yn[:, :DH], yn[:, DH:]], axis=0)


def _mlp_final_body(h_ref, a_ref, w1_ref, b1_ref, w2_ref, b2_ref, o_ref):
    x = _unstack(h_ref[...]) * (1.0 + EPS) + a_ref[...]
    t = jnp.maximum(
        jnp.dot(x, w1_ref[...], preferred_element_type=jnp.float32)
        + b1_ref[...], 0.0)
    o_ref[...] = (jnp.dot(t, w2_ref[...], preferred_element_type=jnp.float32)
                  + b2_ref[...])


_mlp_bn = pl.pallas_call(
    _mlp_bn_body,
    out_shape=jax.ShapeDtypeStruct((2 * NN, DH), jnp.float32),
)

_mlp_final = pl.pallas_call(
    _mlp_final_body,
    out_shape=jax.ShapeDtypeStruct((NN, DD), jnp.float32),
)


def kernel(x, edge_index, params):
    src = edge_index[0]
    dst = edge_index[1]
    src2 = jnp.concatenate([src, src + NN])
    zeros = jnp.zeros((NN, DD), jnp.float32)
    h = jnp.concatenate([x[:, :DH], x[:, DH:]], axis=0)
    for i in range(LAYERS):
        W1, b1, W2, b2 = params["convs"][i]
        agg = _sc_agg(h, src2, dst, zeros)
        b1r = b1.reshape(1, DD)
        b2r = b2.reshape(1, DD)
        if i < LAYERS - 1:
            gamma, beta = params["bns"][i]
            h = _mlp_bn(h, agg, W1, b1r, W2, b2r,
                        gamma.reshape(1, DD), beta.reshape(1, DD))
        else:
            h = _mlp_final(h, agg, W1, b1r, W2, b2r)
    return h
